# Initial kernel scaffold; baseline (speedup 1.0000x reference)
import jax
import jax.numpy as jnp
from jax.experimental import pallas as pl


def kernel(input, target, gt_boxes):
    raise NotImplementedError("write your pallas kernel here")


# speedup vs baseline: 1.3530x; 1.3530x over previous
import jax
import jax.numpy as jnp
from jax.experimental import pallas as pl
from jax.experimental.pallas import tpu as pltpu

_B, _C, _H, _W = 8, 192, 224, 224
_HT = 16
_NBOX = 20


def _body(boxes_ref, inp_ref, tgt_ref, s_ref, n_ref):
    b = pl.program_id(0)
    hi = pl.program_id(1)

    @pl.when((b == 0) & (hi == 0))
    def _init():
        s_ref[0, 0] = 0.0
        n_ref[0, 0] = 0.0

    inp = inp_ref[0]
    tgt = tgt_ref[0]
    tgt2 = jnp.where(jnp.isnan(tgt), inp, tgt)
    diff = inp - tgt2
    l2 = jnp.sum(diff * diff, axis=0)
    anyz = jnp.any(tgt != 0, axis=0)

    ys = hi * _HT + jax.lax.broadcasted_iota(jnp.int32, (_HT, _W), 0)
    xs = jax.lax.broadcasted_iota(jnp.int32, (_HT, _W), 1)
    m = jnp.zeros((_HT, _W), dtype=jnp.bool_)
    for nbx in range(_NBOX):
        x1 = boxes_ref[b, nbx, 0]
        y1 = boxes_ref[b, nbx, 1]
        x2 = boxes_ref[b, nbx, 2]
        y2 = boxes_ref[b, nbx, 3]
        m = m | ((ys >= y1) & (ys < y2) & (xs >= x2) & (xs < x1))

    pos = (anyz & m).astype(jnp.float32)
    s_ref[0, 0] += jnp.sum(pos * l2)
    n_ref[0, 0] += jnp.sum(pos)


def kernel(input, target, gt_boxes):
    s, n = pl.pallas_call(
        _body,
        grid=(_B, _H // _HT),
        in_specs=[
            pl.BlockSpec(memory_space=pltpu.SMEM),
            pl.BlockSpec((1, _C, _HT, _W), lambda b, h: (b, 0, h, 0)),
            pl.BlockSpec((1, _C, _HT, _W), lambda b, h: (b, 0, h, 0)),
        ],
        out_specs=[
            pl.BlockSpec(memory_space=pltpu.SMEM),
            pl.BlockSpec(memory_space=pltpu.SMEM),
        ],
        out_shape=[
            jax.ShapeDtypeStruct((1, 1), jnp.float32),
            jax.ShapeDtypeStruct((1, 1), jnp.float32),
        ],
    )(gt_boxes.astype(jnp.int32), input, target)
    return (0.5 * s[0, 0] / n[0, 0]) / (_C * _B)
